# Initial kernel scaffold; baseline (speedup 1.0000x reference)
import jax
import jax.numpy as jnp
from jax.experimental import pallas as pl


def kernel(scores):
    raise NotImplementedError("write your pallas kernel here")


# speedup vs baseline: 347.0913x; 347.0913x over previous
import functools

import jax
import jax.numpy as jnp
from jax import lax
from jax.experimental import pallas as pl
from jax.experimental.pallas import tpu as pltpu
from jax.experimental.pallas import tpu_sc as plsc

B = 4096
K = 10
L = 16
NC = 2
NS = 16
NW = NC * NS
ROWS_PER_W = B // NW
RCH = 8
NCH = ROWS_PER_W // RCH
CHUNKS = B // L
NEG = -3.0e38


def _max_splat(x, lane):
    for k in (1, 2, 4, 8):
        x = jnp.maximum(x, x.at[lane ^ k].get(mode="promise_in_bounds"))
    return x


def _row_topk(buf, row_off):
    init = tuple(jnp.full((L,), NEG, jnp.float32) for _ in range(K))

    def ins(c, ts):
        v = buf[pl.ds(row_off + c * L, L)]
        out = []
        for t in ts:
            hi = jnp.maximum(t, v)
            v = jnp.minimum(t, v)
            out.append(hi)
        return tuple(out)

    return lax.fori_loop(0, CHUNKS, ins, init, unroll=2)


@functools.partial(
    pl.kernel,
    out_type=jax.ShapeDtypeStruct((B * L,), jnp.float32),
    mesh=plsc.VectorSubcoreMesh(core_axis_name="c", subcore_axis_name="s"),
    scratch_types=[
        pltpu.VMEM((RCH * B,), jnp.float32),
        pltpu.VMEM((RCH * L,), jnp.float32),
    ],
)
def _sc_topk(scores_hbm, out_hbm, buf, stage):
    cid = lax.axis_index("c")
    sid = lax.axis_index("s")
    wid = sid * NC + cid
    row0 = wid * ROWS_PER_W
    lane = lax.iota(jnp.int32, L)

    def chunk(cb, _):
        base_row = row0 + cb * RCH
        pltpu.sync_copy(scores_hbm.at[pl.ds(base_row * B, RCH * B)], buf)

        def row(j, _2):
            ts = list(_row_topk(buf, j * B))
            acc = jnp.zeros((L,), jnp.float32)
            for r in range(K):
                m = _max_splat(ts[0], lane)
                sel = ts[0] == m
                for s in range(K - 1):
                    ts[s] = jnp.where(sel, ts[s + 1], ts[s])
                ts[K - 1] = jnp.where(sel, NEG, ts[K - 1])
                acc = jnp.where(lane == r, m, acc)
            i_glob = base_row + j
            pv = buf[pl.ds(j * B + (i_glob // L) * L, L)]
            pos = pv.at[jnp.broadcast_to(i_glob % L, (L,))].get(
                mode="promise_in_bounds")
            acc = jnp.where(lane == K, pos, acc)
            stage[pl.ds(j * L, L)] = acc
            return 0

        lax.fori_loop(0, RCH, row, 0)
        pltpu.sync_copy(stage, out_hbm.at[pl.ds(base_row * L, RCH * L)])
        return 0

    lax.fori_loop(0, NCH, chunk, 0)


def _finish_body(x_ref, o_ref):
    x = x_ref[...]
    vals = x[:, :K]
    pos = x[:, K:K + 1]
    diff = pos - vals
    sig = 1.0 / (1.0 + jnp.exp(-diff))
    bpr = -jnp.log(jnp.maximum(sig, 1e-8))
    col = lax.broadcasted_iota(jnp.int32, (B, K), 1).astype(jnp.float32)
    w = 1.0 / jnp.log2(col + 2.0)
    g = jnp.sum((vals > pos).astype(jnp.float32), axis=1)
    diag_in = (pos[:, 0] >= vals[:, K - 1]).astype(jnp.float32)
    diag_term = diag_in * (0.6931471805599453 / jnp.log2(g + 2.0))
    row_sum = jnp.sum(w * bpr, axis=1) - diag_term
    total = jnp.sum(row_sum)
    cnt = jnp.float32(K) * B - jnp.sum(diag_in)
    o_ref[0, 0] = total / jnp.maximum(cnt, 1.0)


def _finish(packed):
    return pl.pallas_call(
        _finish_body,
        out_shape=jax.ShapeDtypeStruct((1, 1), jnp.float32),
        out_specs=pl.BlockSpec(memory_space=pltpu.SMEM),
    )(packed)


def kernel(scores):
    packed = _sc_topk(scores.reshape(-1))
    loss = _finish(packed.reshape(B, L))
    return loss[0, 0]


# speedup vs baseline: 438.5867x; 1.2636x over previous
import functools

import jax
import jax.numpy as jnp
from jax import lax
from jax.experimental import pallas as pl
from jax.experimental.pallas import tpu as pltpu
from jax.experimental.pallas import tpu_sc as plsc

B = 4096
K = 10
L = 16
NC = 2
NS = 16
NW = NC * NS
ROWS_PER_W = B // NW
RCH = 8
NCH = ROWS_PER_W // RCH
CHUNKS = B // L
NEG = -3.0e38


def _max_splat(x, lane):
    for k in (1, 2, 4, 8):
        x = jnp.maximum(x, x.at[lane ^ k].get(mode="promise_in_bounds"))
    return x


def _row_topk(buf, j):
    init = tuple(jnp.full((L,), NEG, jnp.float32) for _ in range(K))

    def ins(c, ts):
        v = buf[j, pl.ds(c * L, L)]
        out = []
        for t in ts:
            hi = jnp.maximum(t, v)
            v = jnp.minimum(t, v)
            out.append(hi)
        return tuple(out)

    return lax.fori_loop(0, CHUNKS, ins, init, unroll=2)


@functools.partial(
    pl.kernel,
    out_type=jax.ShapeDtypeStruct((B, L), jnp.float32),
    mesh=plsc.VectorSubcoreMesh(core_axis_name="c", subcore_axis_name="s"),
    scratch_types=[
        pltpu.VMEM((RCH, B), jnp.float32),
        pltpu.VMEM((RCH, L), jnp.float32),
    ],
)
def _sc_topk(scores_hbm, out_hbm, buf, stage):
    cid = lax.axis_index("c")
    sid = lax.axis_index("s")
    wid = sid * NC + cid
    row0 = wid * ROWS_PER_W
    lane = lax.iota(jnp.int32, L)

    def chunk(cb, _):
        base_row = row0 + cb * RCH
        pltpu.sync_copy(scores_hbm.at[pl.ds(base_row, RCH)], buf)

        def row(j, _2):
            ts = list(_row_topk(buf, j))
            acc = jnp.zeros((L,), jnp.float32)
            for r in range(K):
                m = _max_splat(ts[0], lane)
                sel = ts[0] == m
                for s in range(K - 1):
                    ts[s] = jnp.where(sel, ts[s + 1], ts[s])
                ts[K - 1] = jnp.where(sel, NEG, ts[K - 1])
                acc = jnp.where(lane == r, m, acc)
            i_glob = base_row + j
            pv = buf[j, pl.ds((i_glob // L) * L, L)]
            pos = pv.at[jnp.broadcast_to(i_glob % L, (L,))].get(
                mode="promise_in_bounds")
            acc = jnp.where(lane == K, pos, acc)
            stage[j, :] = acc
            return 0

        lax.fori_loop(0, RCH, row, 0)
        pltpu.sync_copy(stage, out_hbm.at[pl.ds(base_row, RCH)])
        return 0

    lax.fori_loop(0, NCH, chunk, 0)


def _finish_body(x_ref, o_ref):
    x = x_ref[...]
    vals = x[:, :K]
    pos = x[:, K:K + 1]
    diff = pos - vals
    sig = 1.0 / (1.0 + jnp.exp(-diff))
    bpr = -jnp.log(jnp.maximum(sig, 1e-8))
    col = lax.broadcasted_iota(jnp.int32, (B, K), 1).astype(jnp.float32)
    w = 1.0 / jnp.log2(col + 2.0)
    g = jnp.sum((vals > pos).astype(jnp.float32), axis=1)
    diag_in = (pos[:, 0] >= vals[:, K - 1]).astype(jnp.float32)
    diag_term = diag_in * (0.6931471805599453 / jnp.log2(g + 2.0))
    row_sum = jnp.sum(w * bpr, axis=1) - diag_term
    total = jnp.sum(row_sum)
    cnt = jnp.float32(K) * B - jnp.sum(diag_in)
    o_ref[0, 0] = total / jnp.maximum(cnt, 1.0)


def _finish(packed):
    return pl.pallas_call(
        _finish_body,
        out_shape=jax.ShapeDtypeStruct((1, 1), jnp.float32),
        out_specs=pl.BlockSpec(memory_space=pltpu.SMEM),
    )(packed)


def kernel(scores):
    packed = _sc_topk(scores)
    loss = _finish(packed)
    return loss[0, 0]


# speedup vs baseline: 520.5425x; 1.1869x over previous
import functools

import jax
import jax.numpy as jnp
from jax import lax
from jax.experimental import pallas as pl
from jax.experimental.pallas import tpu as pltpu
from jax.experimental.pallas import tpu_sc as plsc

B = 4096
K = 10
L = 16
NC = 2
NS = 16
NW = NC * NS
ROWS_PER_W = B // NW
RCH = 8
NCH = ROWS_PER_W // RCH
CHUNKS = B // L
NEG = -3.0e38


def _max_splat(x, lane):
    for k in (1, 2, 4, 8):
        x = jnp.maximum(x, x.at[lane ^ k].get(mode="promise_in_bounds"))
    return x


def _row_topk(buf, j):
    init = tuple(jnp.full((L,), NEG, jnp.float32) for _ in range(K))

    def ins(c, ts):
        v = buf[j, pl.ds(c * L, L)]
        out = []
        for t in ts:
            hi = jnp.maximum(t, v)
            v = jnp.minimum(t, v)
            out.append(hi)
        return tuple(out)

    return lax.fori_loop(0, CHUNKS, ins, init, unroll=2)


@functools.partial(
    pl.kernel,
    out_type=jax.ShapeDtypeStruct((B, L), jnp.float32),
    mesh=plsc.VectorSubcoreMesh(core_axis_name="c", subcore_axis_name="s"),
    scratch_types=[
        pltpu.VMEM((RCH, B), jnp.float32),
        pltpu.VMEM((RCH, B), jnp.float32),
        pltpu.VMEM((RCH, L), jnp.float32),
        pltpu.VMEM((RCH, L), jnp.float32),
        pltpu.SemaphoreType.DMA,
        pltpu.SemaphoreType.DMA,
        pltpu.SemaphoreType.DMA,
        pltpu.SemaphoreType.DMA,
    ],
)
def _sc_topk(scores_hbm, out_hbm, buf0, buf1, st0, st1,
             isem0, isem1, osem0, osem1):
    cid = lax.axis_index("c")
    sid = lax.axis_index("s")
    wid = sid * NC + cid
    row0 = wid * ROWS_PER_W
    lane = lax.iota(jnp.int32, L)

    def icp(cb, bf, sem):
        return pltpu.make_async_copy(
            scores_hbm.at[pl.ds(row0 + cb * RCH, RCH)], bf, sem)

    def ocp(cb, st, sem):
        return pltpu.make_async_copy(
            st, out_hbm.at[pl.ds(row0 + cb * RCH, RCH)], sem)

    def compute_rows(bf, st, base_row):
        def row(j, _2):
            ts = list(_row_topk(bf, j))
            acc = jnp.zeros((L,), jnp.float32)
            for r in range(K):
                m = _max_splat(ts[0], lane)
                sel = ts[0] == m
                for s in range(K - 1):
                    ts[s] = jnp.where(sel, ts[s + 1], ts[s])
                ts[K - 1] = jnp.where(sel, NEG, ts[K - 1])
                acc = jnp.where(lane == r, m, acc)
            i_glob = base_row + j
            pv = bf[j, pl.ds((i_glob // L) * L, L)]
            pos = pv.at[jnp.broadcast_to(i_glob % L, (L,))].get(
                mode="promise_in_bounds")
            acc = jnp.where(lane == K, pos, acc)
            st[j, :] = acc
            return 0

        lax.fori_loop(0, RCH, row, 0)

    icp(0, buf0, isem0).start()

    def halfpair(h, _):
        cb0 = 2 * h
        icp(cb0 + 1, buf1, isem1).start()
        icp(cb0, buf0, isem0).wait()

        @pl.when(h > 0)
        def _():
            ocp(cb0 - 2, st0, osem0).wait()

        compute_rows(buf0, st0, row0 + cb0 * RCH)
        ocp(cb0, st0, osem0).start()

        @pl.when(h + 1 < NCH // 2)
        def _():
            icp(cb0 + 2, buf0, isem0).start()

        icp(cb0 + 1, buf1, isem1).wait()

        @pl.when(h > 0)
        def _():
            ocp(cb0 - 1, st1, osem1).wait()

        compute_rows(buf1, st1, row0 + (cb0 + 1) * RCH)
        ocp(cb0 + 1, st1, osem1).start()
        return 0

    lax.fori_loop(0, NCH // 2, halfpair, 0)
    ocp(NCH - 2, st0, osem0).wait()
    ocp(NCH - 1, st1, osem1).wait()


def _finish_body(x_ref, o_ref):
    x = x_ref[...]
    vals = x[:, :K]
    pos = x[:, K:K + 1]
    diff = pos - vals
    sig = 1.0 / (1.0 + jnp.exp(-diff))
    bpr = -jnp.log(jnp.maximum(sig, 1e-8))
    col = lax.broadcasted_iota(jnp.int32, (B, K), 1).astype(jnp.float32)
    w = 1.0 / jnp.log2(col + 2.0)
    g = jnp.sum((vals > pos).astype(jnp.float32), axis=1)
    diag_in = (pos[:, 0] >= vals[:, K - 1]).astype(jnp.float32)
    diag_term = diag_in * (0.6931471805599453 / jnp.log2(g + 2.0))
    row_sum = jnp.sum(w * bpr, axis=1) - diag_term
    total = jnp.sum(row_sum)
    cnt = jnp.float32(K) * B - jnp.sum(diag_in)
    o_ref[0, 0] = total / jnp.maximum(cnt, 1.0)


def _finish(packed):
    return pl.pallas_call(
        _finish_body,
        out_shape=jax.ShapeDtypeStruct((1, 1), jnp.float32),
        out_specs=pl.BlockSpec(memory_space=pltpu.SMEM),
    )(packed)


def kernel(scores):
    packed = _sc_topk(scores)
    loss = _finish(packed)
    return loss[0, 0]
